# Initial kernel scaffold; baseline (speedup 1.0000x reference)
import jax
import jax.numpy as jnp
from jax.experimental import pallas as pl


def kernel(entity_embed, relation_embed, h_trans_w1, h_trans_w2, h_bias_b, r_trans_w1, r_trans_w2, r_bias_b, sem_trans_w, h_batch, t_batch, r_batch):
    raise NotImplementedError("write your pallas kernel here")


# speedup vs baseline: 9.9428x; 9.9428x over previous
import functools

import jax
import jax.numpy as jnp
from jax import lax
from jax.experimental import pallas as pl
from jax.experimental.pallas import tpu as pltpu
from jax.experimental.pallas import tpu_sc as plsc

E = 4096
D = 128
NREL = 64

_NC = 2
_NS = 16
_NW = _NC * _NS
_BPW = E // _NW


def _sc_gather(table, h_idx, t_idx):
  mesh = plsc.VectorSubcoreMesh(
      core_axis_name="c", subcore_axis_name="s",
      num_cores=_NC, num_subcores=_NS)

  @functools.partial(
      pl.kernel,
      out_type=(jax.ShapeDtypeStruct((E, D), jnp.float32),
                jax.ShapeDtypeStruct((E, D), jnp.float32)),
      mesh=mesh,
      scratch_types=(pltpu.VMEM((_BPW,), jnp.int32),
                     pltpu.VMEM((_BPW, D), jnp.float32),
                     pltpu.VMEM((_BPW,), jnp.int32),
                     pltpu.VMEM((_BPW, D), jnp.float32),
                     pltpu.SemaphoreType.DMA,
                     pltpu.SemaphoreType.DMA),
  )
  def k(table_hbm, hi_hbm, ti_hbm, h_out, t_out,
        hi_v, hrows_v, ti_v, trows_v, sem_h, sem_t):
    wid = lax.axis_index("s") * _NC + lax.axis_index("c")
    base = wid * _BPW
    pltpu.sync_copy(hi_hbm.at[pl.ds(base, _BPW)], hi_v)
    pltpu.sync_copy(ti_hbm.at[pl.ds(base, _BPW)], ti_v)
    ch = pltpu.async_copy(table_hbm.at[hi_v], hrows_v, sem_h)
    ct = pltpu.async_copy(table_hbm.at[ti_v], trows_v, sem_t)
    ch.wait()
    pltpu.sync_copy(hrows_v, h_out.at[pl.ds(base, _BPW)])
    ct.wait()
    pltpu.sync_copy(trows_v, t_out.at[pl.ds(base, _BPW)])

  return k(table, h_idx, t_idx)


def _tc_body(h_ref, t_ref, rel_ref, ridx_ref, w1h_ref, w2h_ref,
             w1r_ref, w2r_ref, bh_ref, br_ref, semw_ref, out_ref):
  mm = lambda a, b: jnp.dot(a, b, preferred_element_type=jnp.float32,
                            precision=lax.Precision.HIGHEST)
  H = h_ref[...]
  T = t_ref[...]
  R = rel_ref[...]
  iota = lax.broadcasted_iota(jnp.int32, (E, NREL), 1)
  onehot = (iota == ridx_ref[...]).astype(jnp.float32)
  Rg = mm(onehot, R)
  Wt = semw_ref[:D, :]
  Wb = semw_ref[D:, :]
  HT = mm(H, Wt)
  HB = mm(H, Wb)
  RT = mm(Rg, Wt)
  RB = mm(Rg, Wb)
  s1 = jnp.sum(Rg * w1h_ref[...], axis=1, keepdims=True)
  s2 = jnp.sum(H * w2r_ref[...], axis=1, keepdims=True)
  s3 = jnp.sum(Rg * w2h_ref[...], axis=1, keepdims=True)
  s4 = jnp.sum(H * w1r_ref[...], axis=1, keepdims=True)
  cvec = mm(bh_ref[...], Wt) + mm(br_ref[...], Wb)
  proj = s1 * HT + s2 * RT + s3 * HB + s4 * RB + cvec
  x = proj * T
  y = jnp.where(x >= 0, x, 0.01 * x)
  out_ref[...] = jnp.sum(y, axis=1, keepdims=True)


def kernel(entity_embed, relation_embed, h_trans_w1, h_trans_w2, h_bias_b,
           r_trans_w1, r_trans_w2, r_bias_b, sem_trans_w,
           h_batch, t_batch, r_batch):
  H, T = _sc_gather(entity_embed, h_batch, t_batch)
  out = pl.pallas_call(
      _tc_body,
      out_shape=jax.ShapeDtypeStruct((E, 1), jnp.float32),
  )(H, T, relation_embed, r_batch.reshape(E, 1),
    h_trans_w1.reshape(1, D), h_trans_w2.reshape(1, D),
    r_trans_w1.reshape(1, D), r_trans_w2.reshape(1, D),
    h_bias_b.reshape(1, D), r_bias_b.reshape(1, D), sem_trans_w)
  return out.reshape(E)
